# Initial kernel scaffold; baseline (speedup 1.0000x reference)
import jax
import jax.numpy as jnp
from jax.experimental import pallas as pl


def kernel(token_ids, weight):
    raise NotImplementedError("write your pallas kernel here")


# speedup vs baseline: 2.9717x; 2.9717x over previous
import functools

import jax
import jax.numpy as jnp
from jax import lax
from jax.experimental import pallas as pl
from jax.experimental.pallas import tpu as pltpu
from jax.experimental.pallas import tpu_sc as plsc

D = 128
CHUNK = 128


def _sc_geometry():
    try:
        info = plsc.get_sparse_core_info()
        return info.num_cores, info.num_subcores
    except Exception:
        return 2, 16


@functools.cache
def _build(B):
    NC, NS = _sc_geometry()
    NW = NC * NS
    assert B % (NW * CHUNK) == 0
    n_chunks = B // (NW * CHUNK)
    mesh = plsc.VectorSubcoreMesh(core_axis_name="c", subcore_axis_name="s")

    @functools.partial(
        pl.kernel,
        out_type=jax.ShapeDtypeStruct((B, D), jnp.float32),
        mesh=mesh,
        scratch_types=[
            pltpu.VMEM((n_chunks, CHUNK), jnp.int32),
            pltpu.VMEM((CHUNK, D), jnp.float32),
            pltpu.SemaphoreType.DMA,
        ],
    )
    def emb(idx_hbm, table_hbm, out_hbm, idx_v, rows_v, sem):
        wid = lax.axis_index("s") * NC + lax.axis_index("c")
        base_chunk = wid * n_chunks
        pltpu.sync_copy(idx_hbm.at[wid], idx_v)

        @pl.loop(0, n_chunks)
        def _(j):
            pltpu.async_copy(table_hbm.at[idx_v.at[j]], rows_v, sem).wait()
            pltpu.sync_copy(
                rows_v, out_hbm.at[pl.ds((base_chunk + j) * CHUNK, CHUNK)]
            )

    return emb


def kernel(token_ids, weight):
    B = token_ids.size
    NC, NS = _sc_geometry()
    NW = NC * NS
    idx = token_ids.reshape(NW, B // (NW * CHUNK), CHUNK)
    out = _build(B)(idx, weight)
    return out.reshape(*token_ids.shape, D)


# speedup vs baseline: 3.3435x; 1.1251x over previous
import functools

import jax
import jax.numpy as jnp
from jax import lax
from jax.experimental import pallas as pl
from jax.experimental.pallas import tpu as pltpu
from jax.experimental.pallas import tpu_sc as plsc

D = 128
CHUNK = 128
NBUF = 5
LOOKAHEAD = 2


def _sc_geometry():
    try:
        info = plsc.get_sparse_core_info()
        return info.num_cores, info.num_subcores
    except Exception:
        return 2, 16


@functools.cache
def _build(B):
    NC, NS = _sc_geometry()
    NW = NC * NS
    assert B % (NW * CHUNK) == 0
    n_chunks = B // (NW * CHUNK)
    assert n_chunks % NBUF == 0
    mesh = plsc.VectorSubcoreMesh(core_axis_name="c", subcore_axis_name="s")

    @functools.partial(
        pl.kernel,
        out_type=jax.ShapeDtypeStruct((B, D), jnp.float32),
        mesh=mesh,
        scratch_types=[
            pltpu.VMEM((n_chunks, CHUNK), jnp.int32),
            [pltpu.VMEM((CHUNK, D), jnp.float32) for _ in range(NBUF)],
            [pltpu.SemaphoreType.DMA for _ in range(NBUF)],
            [pltpu.SemaphoreType.DMA for _ in range(NBUF)],
        ],
    )
    def emb(idx_hbm, table_hbm, out_hbm, idx_v, rows, gsem, wsem):
        wid = lax.axis_index("s") * NC + lax.axis_index("c")
        base_chunk = wid * n_chunks
        pltpu.sync_copy(idx_hbm.at[wid], idx_v)

        def start_gather(chunk, b):
            pltpu.async_copy(table_hbm.at[idx_v.at[chunk]], rows[b], gsem[b])

        def start_write(chunk, b):
            pltpu.async_copy(
                rows[b],
                out_hbm.at[pl.ds((base_chunk + chunk) * CHUNK, CHUNK)],
                wsem[b],
            )

        def wait_write(b):
            pltpu.make_async_copy(
                rows[b], out_hbm.at[pl.ds(0, CHUNK)], wsem[b]
            ).wait()

        def wait_gather(chunk, b):
            pltpu.make_async_copy(
                table_hbm.at[idx_v.at[chunk]], rows[b], gsem[b]
            ).wait()

        for p in range(LOOKAHEAD):
            start_gather(p, p)

        @pl.loop(0, n_chunks, step=NBUF)
        def _(j0):
            for b in range(NBUF):
                j = j0 + b
                g = j + LOOKAHEAD
                bg = (b + LOOKAHEAD) % NBUF

                @pl.when(g < n_chunks)
                def _():
                    @pl.when(j >= NBUF - LOOKAHEAD)
                    def _():
                        wait_write(bg)

                    start_gather(g, bg)

                wait_gather(j, b)
                start_write(j, b)

        for b in range(NBUF):
            wait_write(b)

    return emb


def kernel(token_ids, weight):
    B = token_ids.size
    NC, NS = _sc_geometry()
    NW = NC * NS
    idx = token_ids.reshape(NW, B // (NW * CHUNK), CHUNK)
    out = _build(B)(idx, weight)
    return out.reshape(*token_ids.shape, D)


# speedup vs baseline: 5.8862x; 1.7605x over previous
import functools

import jax
import jax.numpy as jnp
from jax import lax
from jax.experimental import pallas as pl
from jax.experimental.pallas import tpu as pltpu
from jax.experimental.pallas import tpu_sc as plsc

NBUF = 4
LOOKAHEAD = 2


def _sc_geometry():
    try:
        info = plsc.get_sparse_core_info()
        return info.num_cores, info.num_subcores
    except Exception:
        return 2, 16


@functools.cache
def _build(B, L, D):
    NC, NS = _sc_geometry()
    NW = NC * NS
    assert B % (NW * NBUF) == 0 and L <= 128
    n_rows = B // NW
    mesh = plsc.VectorSubcoreMesh(core_axis_name="c", subcore_axis_name="s")

    @functools.partial(
        pl.kernel,
        out_type=jax.ShapeDtypeStruct((B, L, D), jnp.float32),
        mesh=mesh,
        scratch_types=[
            pltpu.VMEM((n_rows, L), jnp.int32),
            [pltpu.VMEM((L, D), jnp.float32) for _ in range(NBUF)],
            [pltpu.SemaphoreType.DMA for _ in range(NBUF)],
            [pltpu.SemaphoreType.DMA for _ in range(NBUF)],
        ],
    )
    def emb(idx_hbm, table_hbm, out_hbm, idx_v, rows, gsem, wsem):
        wid = lax.axis_index("s") * NC + lax.axis_index("c")
        base = wid * n_rows
        pltpu.sync_copy(idx_hbm.at[pl.ds(base, n_rows)], idx_v)

        def start_gather(r, b):
            pltpu.async_copy(table_hbm.at[idx_v.at[r]], rows[b], gsem[b])

        def start_write(r, b):
            pltpu.async_copy(rows[b], out_hbm.at[base + r], wsem[b])

        def wait_write(b):
            pltpu.make_async_copy(rows[b], out_hbm.at[0], wsem[b]).wait()

        def wait_gather(r, b):
            pltpu.make_async_copy(
                table_hbm.at[idx_v.at[r]], rows[b], gsem[b]
            ).wait()

        for p in range(LOOKAHEAD):
            start_gather(p, p)

        @pl.loop(0, n_rows, step=NBUF)
        def _(j0):
            for b in range(NBUF):
                j = j0 + b
                g = j + LOOKAHEAD
                bg = (b + LOOKAHEAD) % NBUF

                @pl.when(g < n_rows)
                def _():
                    @pl.when(j >= NBUF - LOOKAHEAD)
                    def _():
                        wait_write(bg)

                    start_gather(g, bg)

                wait_gather(j, b)
                start_write(j, b)

        for b in range(NBUF):
            wait_write(b)

    return emb


def kernel(token_ids, weight):
    B, L = token_ids.shape
    D = weight.shape[1]
    return _build(B, L, D)(token_ids, weight)
